# Initial kernel scaffold; baseline (speedup 1.0000x reference)
import jax
import jax.numpy as jnp
from jax.experimental import pallas as pl


def kernel(x, edge_index, W1, b1, W2, b2):
    raise NotImplementedError("write your pallas kernel here")


# speedup vs baseline: 29.7920x; 29.7920x over previous
import functools

import jax
import jax.numpy as jnp
from jax import lax
from jax.experimental import pallas as pl
from jax.experimental.pallas import tpu as pltpu
from jax.experimental.pallas import tpu_sc as plsc

N = 10000
E = 320000
D_IN = 128
D = 16

NC = 2
NS = 16
NW = NC * NS
CH = 128
EPW = 10240
NCH = EPW // CH
EPAD = NW * EPW
ACC_ROWS = 10112
RPS = ACC_ROWS // NS

_sc_mesh = plsc.VectorSubcoreMesh(core_axis_name="c", subcore_axis_name="s")
_sc_params = pltpu.CompilerParams(use_tc_tiling_on_sc=False)


@functools.partial(
    pl.kernel,
    out_type=jax.ShapeDtypeStruct((NC, ACC_ROWS, D), jnp.float32),
    mesh=_sc_mesh,
    scratch_types=[
        pltpu.VMEM((NCH, CH), jnp.int32),
        pltpu.VMEM((CH, D), jnp.float32),
        pltpu.VMEM((RPS, D), jnp.float32),
        pltpu.VMEM_SHARED((ACC_ROWS, D), jnp.float32),
    ],
    compiler_params=_sc_params,
)
def _deg_sc(dst_hbm, out_hbm, didx, ones, stage, acc):
    c = lax.axis_index("c")
    s = lax.axis_index("s")
    wid = c * NS + s
    z16 = jnp.zeros((D,), jnp.float32)
    o16 = jnp.ones((D,), jnp.float32)

    def fill(i, _):
        stage[i, :] = z16
        return 0

    lax.fori_loop(0, RPS, fill, 0)

    def fill1(i, _):
        ones[i, :] = o16
        return 0

    lax.fori_loop(0, CH, fill1, 0)
    pltpu.sync_copy(stage, acc.at[pl.ds(s * RPS, RPS)])
    plsc.subcore_barrier()

    pltpu.sync_copy(dst_hbm.at[pl.ds(wid * NCH, NCH)], didx)

    def body(i, _):
        pltpu.sync_copy(ones, acc.at[didx.at[i]], add=True)
        return 0

    lax.fori_loop(0, NCH, body, 0)
    plsc.subcore_barrier()
    r0 = s * RPS
    pltpu.sync_copy(acc.at[pl.ds(r0, RPS)], out_hbm.at[c, pl.ds(r0, RPS)])


@functools.partial(
    pl.kernel,
    out_type=jax.ShapeDtypeStruct((NC, ACC_ROWS, D), jnp.float32),
    mesh=_sc_mesh,
    scratch_types=[
        pltpu.VMEM((NCH, CH), jnp.int32),
        pltpu.VMEM((NCH, CH), jnp.int32),
        pltpu.VMEM((CH, D), jnp.float32),
        pltpu.VMEM((RPS, D), jnp.float32),
        pltpu.VMEM_SHARED((ACC_ROWS, D), jnp.float32),
        pltpu.SemaphoreType.DMA,
    ],
    compiler_params=_sc_params,
)
def _agg_sc(s_hbm, src_hbm, dst_hbm, out_hbm, sidx, didx, rows, stage, acc, sem):
    c = lax.axis_index("c")
    s = lax.axis_index("s")
    wid = c * NS + s
    z16 = jnp.zeros((D,), jnp.float32)

    def fill(i, _):
        stage[i, :] = z16
        return 0

    lax.fori_loop(0, RPS, fill, 0)
    pltpu.sync_copy(stage, acc.at[pl.ds(s * RPS, RPS)])
    plsc.subcore_barrier()

    pltpu.sync_copy(src_hbm.at[pl.ds(wid * NCH, NCH)], sidx)
    pltpu.sync_copy(dst_hbm.at[pl.ds(wid * NCH, NCH)], didx)

    def body(i, _):
        pltpu.async_copy(s_hbm.at[sidx.at[i]], rows, sem).wait()
        pltpu.sync_copy(rows, acc.at[didx.at[i]], add=True)
        return 0

    lax.fori_loop(0, NCH, body, 0)
    plsc.subcore_barrier()
    r0 = s * RPS
    pltpu.sync_copy(acc.at[pl.ds(r0, RPS)], out_hbm.at[c, pl.ds(r0, RPS)])


def _prep_body(x_ref, w1_ref, degp_ref, s1_ref, dinv_ref):
    deg = degp_ref[0, :N, 0:1] + degp_ref[1, :N, 0:1] + 1.0
    dinv = lax.rsqrt(deg)
    xw = jnp.dot(x_ref[...], w1_ref[...], preferred_element_type=jnp.float32)
    s1_ref[...] = xw * dinv
    dinv_ref[...] = dinv


def _mid_body(p_ref, s1_ref, dinv_ref, b1_ref, w2_ref, s2_ref):
    dinv = dinv_ref[...]
    agg = p_ref[0, :N, :] + p_ref[1, :N, :] + s1_ref[...]
    h = jnp.maximum(agg * dinv + b1_ref[...], 0.0)
    s2_ref[...] = jnp.dot(h, w2_ref[...], preferred_element_type=jnp.float32) * dinv


def _fin_body(p_ref, s2_ref, dinv_ref, b2_ref, o_ref):
    agg = p_ref[0, :N, :] + p_ref[1, :N, :] + s2_ref[...]
    o_ref[...] = agg * dinv_ref[...] + b2_ref[...]


_prep_tc = pl.pallas_call(
    _prep_body,
    out_shape=[
        jax.ShapeDtypeStruct((N, D), jnp.float32),
        jax.ShapeDtypeStruct((N, 1), jnp.float32),
    ],
)

_mid_tc = pl.pallas_call(
    _mid_body,
    out_shape=jax.ShapeDtypeStruct((N, D), jnp.float32),
)

_fin_tc = pl.pallas_call(
    _fin_body,
    out_shape=jax.ShapeDtypeStruct((N, D), jnp.float32),
)


def kernel(x, edge_index, W1, b1, W2, b2):
    ei = edge_index.astype(jnp.int32)
    pad = EPAD - E
    src2d = jnp.concatenate(
        [ei[0], jnp.zeros((pad,), jnp.int32)]).reshape(NW * NCH, CH)
    dst2d = jnp.concatenate(
        [ei[1], jnp.full((pad,), N, jnp.int32)]).reshape(NW * NCH, CH)

    degp = _deg_sc(dst2d)
    s1, dinv = _prep_tc(x, W1, degp)
    p1 = _agg_sc(s1, src2d, dst2d)
    s2 = _mid_tc(p1, s1, dinv, b1.reshape(1, D), W2)
    p2 = _agg_sc(s2, src2d, dst2d)
    return _fin_tc(p2, s2, dinv, b2.reshape(1, D))


# speedup vs baseline: 38.4891x; 1.2919x over previous
import functools

import jax
import jax.numpy as jnp
from jax import lax
from jax.experimental import pallas as pl
from jax.experimental.pallas import tpu as pltpu
from jax.experimental.pallas import tpu_sc as plsc

N = 10000
E = 320000
D_IN = 128
D = 16

NC = 2
NS = 16
NW = NC * NS
CH = 128
EPW = 10240
NCH = EPW // CH
EPAD = NW * EPW
ACC_ROWS = 10112
RPS = ACC_ROWS // NS

_sc_mesh = plsc.VectorSubcoreMesh(core_axis_name="c", subcore_axis_name="s")
_sc_params = pltpu.CompilerParams(use_tc_tiling_on_sc=False)


@functools.partial(
    pl.kernel,
    out_type=jax.ShapeDtypeStruct((NC, ACC_ROWS, D), jnp.float32),
    mesh=_sc_mesh,
    scratch_types=[
        pltpu.VMEM((NCH, CH), jnp.int32),
        pltpu.VMEM((CH, D), jnp.float32),
        pltpu.VMEM((RPS, D), jnp.float32),
        pltpu.VMEM_SHARED((ACC_ROWS, D), jnp.float32),
    ],
    compiler_params=_sc_params,
)
def _deg_sc(dst_hbm, out_hbm, didx, ones, stage, acc):
    c = lax.axis_index("c")
    s = lax.axis_index("s")
    wid = c * NS + s
    z16 = jnp.zeros((D,), jnp.float32)
    o16 = jnp.ones((D,), jnp.float32)

    def fill(i, _):
        stage[i, :] = z16
        return 0

    lax.fori_loop(0, RPS, fill, 0)

    def fill1(i, _):
        ones[i, :] = o16
        return 0

    lax.fori_loop(0, CH, fill1, 0)
    pltpu.sync_copy(stage, acc.at[pl.ds(s * RPS, RPS)])
    plsc.subcore_barrier()

    pltpu.sync_copy(dst_hbm.at[pl.ds(wid * NCH, NCH)], didx)

    def body(i, _):
        pltpu.sync_copy(ones, acc.at[didx.at[i]], add=True)
        return 0

    lax.fori_loop(0, NCH, body, 0)
    plsc.subcore_barrier()
    r0 = s * RPS
    pltpu.sync_copy(acc.at[pl.ds(r0, RPS)], out_hbm.at[c, pl.ds(r0, RPS)])


@functools.partial(
    pl.kernel,
    out_type=jax.ShapeDtypeStruct((NC, ACC_ROWS, D), jnp.float32),
    mesh=_sc_mesh,
    scratch_types=[
        pltpu.VMEM((NCH, CH), jnp.int32),
        pltpu.VMEM((NCH, CH), jnp.int32),
        pltpu.VMEM((CH, D), jnp.float32),
        pltpu.VMEM((CH, D), jnp.float32),
        pltpu.VMEM((RPS, D), jnp.float32),
        pltpu.VMEM_SHARED((ACC_ROWS, D), jnp.float32),
        pltpu.SemaphoreType.DMA,
        pltpu.SemaphoreType.DMA,
    ],
    compiler_params=_sc_params,
)
def _agg_sc(s_hbm, src_hbm, dst_hbm, out_hbm, sidx, didx, rows0, rows1,
            stage, acc, sem0, sem1):
    c = lax.axis_index("c")
    s = lax.axis_index("s")
    wid = c * NS + s
    z16 = jnp.zeros((D,), jnp.float32)

    def fill(i, _):
        stage[i, :] = z16
        return 0

    lax.fori_loop(0, RPS, fill, 0)
    pltpu.sync_copy(stage, acc.at[pl.ds(s * RPS, RPS)])
    plsc.subcore_barrier()

    pltpu.sync_copy(src_hbm.at[pl.ds(wid * NCH, NCH)], sidx)
    pltpu.sync_copy(dst_hbm.at[pl.ds(wid * NCH, NCH)], didx)

    pltpu.async_copy(s_hbm.at[sidx.at[0]], rows0, sem0)
    pltpu.async_copy(s_hbm.at[sidx.at[1]], rows1, sem1)

    def body(j, _):
        c0 = 2 * j
        c1 = c0 + 1
        pltpu.make_async_copy(s_hbm.at[sidx.at[c0]], rows0, sem0).wait()
        pltpu.sync_copy(rows0, acc.at[didx.at[c0]], add=True)
        pltpu.async_copy(s_hbm.at[sidx.at[c0 + 2]], rows0, sem0)
        pltpu.make_async_copy(s_hbm.at[sidx.at[c1]], rows1, sem1).wait()
        pltpu.sync_copy(rows1, acc.at[didx.at[c1]], add=True)
        pltpu.async_copy(s_hbm.at[sidx.at[c1 + 2]], rows1, sem1)
        return 0

    lax.fori_loop(0, NCH // 2 - 1, body, 0)
    cl = NCH - 2
    pltpu.make_async_copy(s_hbm.at[sidx.at[cl]], rows0, sem0).wait()
    pltpu.sync_copy(rows0, acc.at[didx.at[cl]], add=True)
    pltpu.make_async_copy(s_hbm.at[sidx.at[cl + 1]], rows1, sem1).wait()
    pltpu.sync_copy(rows1, acc.at[didx.at[cl + 1]], add=True)
    plsc.subcore_barrier()
    r0 = s * RPS
    pltpu.sync_copy(acc.at[pl.ds(r0, RPS)], out_hbm.at[c, pl.ds(r0, RPS)])


def _prep_body(x_ref, w1_ref, degp_ref, s1_ref, dinv_ref):
    deg = degp_ref[0, :N, 0:1] + degp_ref[1, :N, 0:1] + 1.0
    dinv = lax.rsqrt(deg)
    xw = jnp.dot(x_ref[...], w1_ref[...], preferred_element_type=jnp.float32)
    s1_ref[...] = xw * dinv
    dinv_ref[...] = dinv


def _mid_body(p_ref, s1_ref, dinv_ref, b1_ref, w2_ref, s2_ref):
    dinv = dinv_ref[...]
    agg = p_ref[0, :N, :] + p_ref[1, :N, :] + s1_ref[...]
    h = jnp.maximum(agg * dinv + b1_ref[...], 0.0)
    s2_ref[...] = jnp.dot(h, w2_ref[...], preferred_element_type=jnp.float32) * dinv


def _fin_body(p_ref, s2_ref, dinv_ref, b2_ref, o_ref):
    agg = p_ref[0, :N, :] + p_ref[1, :N, :] + s2_ref[...]
    o_ref[...] = agg * dinv_ref[...] + b2_ref[...]


_prep_tc = pl.pallas_call(
    _prep_body,
    out_shape=[
        jax.ShapeDtypeStruct((N, D), jnp.float32),
        jax.ShapeDtypeStruct((N, 1), jnp.float32),
    ],
)

_mid_tc = pl.pallas_call(
    _mid_body,
    out_shape=jax.ShapeDtypeStruct((N, D), jnp.float32),
)

_fin_tc = pl.pallas_call(
    _fin_body,
    out_shape=jax.ShapeDtypeStruct((N, D), jnp.float32),
)


def kernel(x, edge_index, W1, b1, W2, b2):
    ei = edge_index.astype(jnp.int32)
    pad = EPAD - E
    src2d = jnp.concatenate(
        [ei[0], jnp.zeros((pad,), jnp.int32)]).reshape(NW * NCH, CH)
    dst2d = jnp.concatenate(
        [ei[1], jnp.full((pad,), N, jnp.int32)]).reshape(NW * NCH, CH)

    degp = _deg_sc(dst2d)
    s1, dinv = _prep_tc(x, W1, degp)
    p1 = _agg_sc(s1, src2d, dst2d)
    s2 = _mid_tc(p1, s1, dinv, b1.reshape(1, D), W2)
    p2 = _agg_sc(s2, src2d, dst2d)
    return _fin_tc(p2, s2, dinv, b2.reshape(1, D))


# speedup vs baseline: 54.9407x; 1.4274x over previous
import functools

import jax
import jax.numpy as jnp
from jax import lax
from jax.experimental import pallas as pl
from jax.experimental.pallas import tpu as pltpu
from jax.experimental.pallas import tpu_sc as plsc

N = 10000
E = 320000
D_IN = 128
D = 16

NC = 2
NS = 16
NW = NC * NS
CH = 128
EPW = 10240
NCH = EPW // CH
EPAD = NW * EPW
ACC_ROWS = 10112
RPS = ACC_ROWS // NS

_sc_mesh = plsc.VectorSubcoreMesh(core_axis_name="c", subcore_axis_name="s")
_sc_params = pltpu.CompilerParams(use_tc_tiling_on_sc=False)


@functools.partial(
    pl.kernel,
    out_type=jax.ShapeDtypeStruct((NC, ACC_ROWS, D), jnp.float32),
    mesh=_sc_mesh,
    scratch_types=[
        pltpu.VMEM((NCH, CH), jnp.int32),
        pltpu.VMEM((CH, D), jnp.float32),
        pltpu.VMEM((RPS, D), jnp.float32),
        pltpu.VMEM_SHARED((ACC_ROWS, D), jnp.float32),
    ],
    compiler_params=_sc_params,
)
def _deg_sc(dst_hbm, out_hbm, didx, ones, stage, acc):
    c = lax.axis_index("c")
    s = lax.axis_index("s")
    wid = c * NS + s
    z16 = jnp.zeros((D,), jnp.float32)
    o16 = jnp.ones((D,), jnp.float32)

    def fill(i, _):
        stage[i, :] = z16
        return 0

    lax.fori_loop(0, RPS, fill, 0)

    def fill1(i, _):
        ones[i, :] = o16
        return 0

    lax.fori_loop(0, CH, fill1, 0)
    pltpu.sync_copy(stage, acc.at[pl.ds(s * RPS, RPS)])
    plsc.subcore_barrier()

    pltpu.sync_copy(dst_hbm.at[pl.ds(wid * NCH, NCH)], didx)

    def body(i, _):
        pltpu.sync_copy(ones, acc.at[didx.at[i]], add=True)
        return 0

    lax.fori_loop(0, NCH, body, 0)
    plsc.subcore_barrier()
    r0 = s * RPS
    pltpu.sync_copy(acc.at[pl.ds(r0, RPS)], out_hbm.at[c, pl.ds(r0, RPS)])


@functools.partial(
    pl.kernel,
    out_type=jax.ShapeDtypeStruct((NC, ACC_ROWS, D), jnp.float32),
    mesh=_sc_mesh,
    scratch_types=[
        pltpu.VMEM((NCH, CH), jnp.int32),
        pltpu.VMEM((NCH, CH), jnp.int32),
        pltpu.VMEM((CH, D), jnp.float32),
        pltpu.VMEM((CH, D), jnp.float32),
        pltpu.VMEM((RPS, D), jnp.float32),
        pltpu.VMEM_SHARED((ACC_ROWS, D), jnp.float32),
        pltpu.VMEM_SHARED((ACC_ROWS, D), jnp.float32),
        pltpu.SemaphoreType.DMA,
        pltpu.SemaphoreType.DMA,
    ],
    compiler_params=_sc_params,
)
def _agg_sc(s_hbm, src_hbm, dst_hbm, out_hbm, sidx, didx, rows0, rows1,
            stage, tbl, acc, sem0, sem1):
    c = lax.axis_index("c")
    s = lax.axis_index("s")
    wid = c * NS + s
    z16 = jnp.zeros((D,), jnp.float32)
    r0 = s * RPS

    pltpu.sync_copy(s_hbm.at[pl.ds(r0, RPS)], tbl.at[pl.ds(r0, RPS)])

    def fill(i, _):
        stage[i, :] = z16
        return 0

    lax.fori_loop(0, RPS, fill, 0)
    pltpu.sync_copy(stage, acc.at[pl.ds(r0, RPS)])

    pltpu.sync_copy(src_hbm.at[pl.ds(wid * NCH, NCH)], sidx)
    pltpu.sync_copy(dst_hbm.at[pl.ds(wid * NCH, NCH)], didx)
    plsc.subcore_barrier()

    pltpu.async_copy(tbl.at[sidx.at[0]], rows0, sem0)
    pltpu.async_copy(tbl.at[sidx.at[1]], rows1, sem1)

    def body(j, _):
        c0 = 2 * j
        c1 = c0 + 1
        pltpu.make_async_copy(tbl.at[sidx.at[c0]], rows0, sem0).wait()
        pltpu.sync_copy(rows0, acc.at[didx.at[c0]], add=True)
        pltpu.async_copy(tbl.at[sidx.at[c0 + 2]], rows0, sem0)
        pltpu.make_async_copy(tbl.at[sidx.at[c1]], rows1, sem1).wait()
        pltpu.sync_copy(rows1, acc.at[didx.at[c1]], add=True)
        pltpu.async_copy(tbl.at[sidx.at[c1 + 2]], rows1, sem1)
        return 0

    lax.fori_loop(0, NCH // 2 - 1, body, 0)
    cl = NCH - 2
    pltpu.make_async_copy(tbl.at[sidx.at[cl]], rows0, sem0).wait()
    pltpu.sync_copy(rows0, acc.at[didx.at[cl]], add=True)
    pltpu.make_async_copy(tbl.at[sidx.at[cl + 1]], rows1, sem1).wait()
    pltpu.sync_copy(rows1, acc.at[didx.at[cl + 1]], add=True)
    plsc.subcore_barrier()
    r0 = s * RPS
    pltpu.sync_copy(acc.at[pl.ds(r0, RPS)], out_hbm.at[c, pl.ds(r0, RPS)])


def _prep_body(x_ref, w1_ref, degp_ref, s1_ref, dinv_ref):
    deg = degp_ref[0, :N, 0:1] + degp_ref[1, :N, 0:1] + 1.0
    dinv = lax.rsqrt(deg)
    xw = jnp.dot(x_ref[...], w1_ref[...], preferred_element_type=jnp.float32)
    s1_ref[:N, :] = xw * dinv
    s1_ref[N:, :] = jnp.zeros((ACC_ROWS - N, D), jnp.float32)
    dinv_ref[...] = dinv


def _mid_body(p_ref, s1_ref, dinv_ref, b1_ref, w2_ref, s2_ref):
    dinv = dinv_ref[...]
    agg = p_ref[0, :N, :] + p_ref[1, :N, :] + s1_ref[:N, :]
    h = jnp.maximum(agg * dinv + b1_ref[...], 0.0)
    s2_ref[:N, :] = jnp.dot(h, w2_ref[...], preferred_element_type=jnp.float32) * dinv
    s2_ref[N:, :] = jnp.zeros((ACC_ROWS - N, D), jnp.float32)


def _fin_body(p_ref, s2_ref, dinv_ref, b2_ref, o_ref):
    agg = p_ref[0, :N, :] + p_ref[1, :N, :] + s2_ref[:N, :]
    o_ref[...] = agg * dinv_ref[...] + b2_ref[...]


_prep_tc = pl.pallas_call(
    _prep_body,
    out_shape=[
        jax.ShapeDtypeStruct((ACC_ROWS, D), jnp.float32),
        jax.ShapeDtypeStruct((N, 1), jnp.float32),
    ],
)

_mid_tc = pl.pallas_call(
    _mid_body,
    out_shape=jax.ShapeDtypeStruct((ACC_ROWS, D), jnp.float32),
)

_fin_tc = pl.pallas_call(
    _fin_body,
    out_shape=jax.ShapeDtypeStruct((N, D), jnp.float32),
)


def kernel(x, edge_index, W1, b1, W2, b2):
    ei = edge_index.astype(jnp.int32)
    pad = EPAD - E
    src2d = jnp.concatenate(
        [ei[0], jnp.zeros((pad,), jnp.int32)]).reshape(NW * NCH, CH)
    dst2d = jnp.concatenate(
        [ei[1], jnp.full((pad,), N, jnp.int32)]).reshape(NW * NCH, CH)

    degp = _deg_sc(dst2d)
    s1, dinv = _prep_tc(x, W1, degp)
    p1 = _agg_sc(s1, src2d, dst2d)
    s2 = _mid_tc(p1, s1, dinv, b1.reshape(1, D), W2)
    p2 = _agg_sc(s2, src2d, dst2d)
    return _fin_tc(p2, s2, dinv, b2.reshape(1, D))


# speedup vs baseline: 55.3923x; 1.0082x over previous
import functools

import jax
import jax.numpy as jnp
from jax import lax
from jax.experimental import pallas as pl
from jax.experimental.pallas import tpu as pltpu
from jax.experimental.pallas import tpu_sc as plsc

N = 10000
E = 320000
D_IN = 128
D = 16

NC = 2
NS = 16
NW = NC * NS
CH = 80
EPW = E // NW
NCH = EPW // CH
DSTOFF = E // CH
ACC_ROWS = 10112
RPS = ACC_ROWS // NS

_sc_mesh = plsc.VectorSubcoreMesh(core_axis_name="c", subcore_axis_name="s")
_sc_params = pltpu.CompilerParams(use_tc_tiling_on_sc=False)


@functools.partial(
    pl.kernel,
    out_type=jax.ShapeDtypeStruct((NC, ACC_ROWS, D), jnp.float32),
    mesh=_sc_mesh,
    scratch_types=[
        pltpu.VMEM((NCH, CH), jnp.int32),
        pltpu.VMEM((CH, D), jnp.float32),
        pltpu.VMEM((RPS, D), jnp.float32),
        pltpu.VMEM_SHARED((ACC_ROWS, D), jnp.float32),
    ],
    compiler_params=_sc_params,
)
def _deg_sc(e2d_hbm, out_hbm, didx, ones, stage, acc):
    c = lax.axis_index("c")
    s = lax.axis_index("s")
    wid = c * NS + s
    z16 = jnp.zeros((D,), jnp.float32)
    o16 = jnp.ones((D,), jnp.float32)

    def fill(i, _):
        stage[i, :] = z16
        return 0

    lax.fori_loop(0, RPS, fill, 0)

    def fill1(i, _):
        ones[i, :] = o16
        return 0

    lax.fori_loop(0, CH, fill1, 0)
    pltpu.sync_copy(stage, acc.at[pl.ds(s * RPS, RPS)])
    pltpu.sync_copy(e2d_hbm.at[pl.ds(DSTOFF + wid * NCH, NCH)], didx)
    plsc.subcore_barrier()

    def body(i, _):
        pltpu.sync_copy(ones, acc.at[didx.at[i]], add=True)
        return 0

    lax.fori_loop(0, NCH, body, 0)
    plsc.subcore_barrier()
    r0 = s * RPS
    pltpu.sync_copy(acc.at[pl.ds(r0, RPS)], out_hbm.at[c, pl.ds(r0, RPS)])


@functools.partial(
    pl.kernel,
    out_type=jax.ShapeDtypeStruct((NC, ACC_ROWS, D), jnp.float32),
    mesh=_sc_mesh,
    scratch_types=[
        pltpu.VMEM((NCH, CH), jnp.int32),
        pltpu.VMEM((NCH, CH), jnp.int32),
        pltpu.VMEM((CH, D), jnp.float32),
        pltpu.VMEM((CH, D), jnp.float32),
        pltpu.VMEM((RPS, D), jnp.float32),
        pltpu.VMEM_SHARED((ACC_ROWS, D), jnp.float32),
        pltpu.VMEM_SHARED((ACC_ROWS, D), jnp.float32),
        pltpu.SemaphoreType.DMA,
        pltpu.SemaphoreType.DMA,
    ],
    compiler_params=_sc_params,
)
def _agg_sc(s_hbm, e2d_hbm, out_hbm, sidx, didx, rows0, rows1,
            stage, tbl, acc, sem0, sem1):
    c = lax.axis_index("c")
    s = lax.axis_index("s")
    wid = c * NS + s
    z16 = jnp.zeros((D,), jnp.float32)
    r0 = s * RPS

    pltpu.sync_copy(s_hbm.at[pl.ds(r0, RPS)], tbl.at[pl.ds(r0, RPS)])

    def fill(i, _):
        stage[i, :] = z16
        return 0

    lax.fori_loop(0, RPS, fill, 0)
    pltpu.sync_copy(stage, acc.at[pl.ds(r0, RPS)])

    pltpu.sync_copy(e2d_hbm.at[pl.ds(wid * NCH, NCH)], sidx)
    pltpu.sync_copy(e2d_hbm.at[pl.ds(DSTOFF + wid * NCH, NCH)], didx)
    plsc.subcore_barrier()

    pltpu.async_copy(tbl.at[sidx.at[0]], rows0, sem0)
    pltpu.async_copy(tbl.at[sidx.at[1]], rows1, sem1)

    def body(j, _):
        c0 = 2 * j
        c1 = c0 + 1
        pltpu.make_async_copy(tbl.at[sidx.at[c0]], rows0, sem0).wait()
        pltpu.sync_copy(rows0, acc.at[didx.at[c0]], add=True)
        pltpu.async_copy(tbl.at[sidx.at[c0 + 2]], rows0, sem0)
        pltpu.make_async_copy(tbl.at[sidx.at[c1]], rows1, sem1).wait()
        pltpu.sync_copy(rows1, acc.at[didx.at[c1]], add=True)
        pltpu.async_copy(tbl.at[sidx.at[c1 + 2]], rows1, sem1)
        return 0

    lax.fori_loop(0, NCH // 2 - 1, body, 0)
    cl = 2 * (NCH // 2 - 1)
    pltpu.make_async_copy(tbl.at[sidx.at[cl]], rows0, sem0).wait()
    pltpu.sync_copy(rows0, acc.at[didx.at[cl]], add=True)
    pltpu.make_async_copy(tbl.at[sidx.at[cl + 1]], rows1, sem1).wait()
    pltpu.sync_copy(rows1, acc.at[didx.at[cl + 1]], add=True)
    pltpu.async_copy(tbl.at[sidx.at[cl + 2]], rows0, sem0).wait()
    pltpu.sync_copy(rows0, acc.at[didx.at[cl + 2]], add=True)

    plsc.subcore_barrier()
    pltpu.sync_copy(acc.at[pl.ds(r0, RPS)], out_hbm.at[c, pl.ds(r0, RPS)])


BN = 1000


def _prep_body(x_ref, w1_ref, degp_ref, s1_ref, dinv_ref):
    deg = degp_ref[0, :, 0:1] + degp_ref[1, :, 0:1] + 1.0
    dinv = lax.rsqrt(deg)
    xw = jnp.dot(x_ref[...], w1_ref[...], preferred_element_type=jnp.float32)
    s1_ref[...] = xw * dinv
    dinv_ref[...] = dinv


def _mid_body(p_ref, s1_ref, dinv_ref, b1_ref, w2_ref, s2_ref):
    dinv = dinv_ref[...]
    agg = p_ref[0] + p_ref[1] + s1_ref[...]
    h = jnp.maximum(agg * dinv + b1_ref[...], 0.0)
    s2_ref[...] = jnp.dot(h, w2_ref[...], preferred_element_type=jnp.float32) * dinv


def _fin_body(p_ref, s2_ref, dinv_ref, b2_ref, o_ref):
    agg = p_ref[0] + p_ref[1] + s2_ref[...]
    o_ref[...] = agg * dinv_ref[...] + b2_ref[...]


_prep_tc = pl.pallas_call(
    _prep_body,
    grid=(N // BN,),
    in_specs=[
        pl.BlockSpec((BN, D_IN), lambda i: (i, 0)),
        pl.BlockSpec((D_IN, D), lambda i: (0, 0)),
        pl.BlockSpec((NC, BN, D), lambda i: (0, i, 0)),
    ],
    out_specs=[
        pl.BlockSpec((BN, D), lambda i: (i, 0)),
        pl.BlockSpec((BN, 1), lambda i: (i, 0)),
    ],
    out_shape=[
        jax.ShapeDtypeStruct((ACC_ROWS, D), jnp.float32),
        jax.ShapeDtypeStruct((N, 1), jnp.float32),
    ],
)

_mid_tc = pl.pallas_call(
    _mid_body,
    grid=(N // BN,),
    in_specs=[
        pl.BlockSpec((NC, BN, D), lambda i: (0, i, 0)),
        pl.BlockSpec((BN, D), lambda i: (i, 0)),
        pl.BlockSpec((BN, 1), lambda i: (i, 0)),
        pl.BlockSpec((1, D), lambda i: (0, 0)),
        pl.BlockSpec((D, D), lambda i: (0, 0)),
    ],
    out_specs=pl.BlockSpec((BN, D), lambda i: (i, 0)),
    out_shape=jax.ShapeDtypeStruct((ACC_ROWS, D), jnp.float32),
)

_fin_tc = pl.pallas_call(
    _fin_body,
    grid=(N // BN,),
    in_specs=[
        pl.BlockSpec((NC, BN, D), lambda i: (0, i, 0)),
        pl.BlockSpec((BN, D), lambda i: (i, 0)),
        pl.BlockSpec((BN, 1), lambda i: (i, 0)),
        pl.BlockSpec((1, D), lambda i: (0, 0)),
    ],
    out_specs=pl.BlockSpec((BN, D), lambda i: (i, 0)),
    out_shape=jax.ShapeDtypeStruct((N, D), jnp.float32),
)


def kernel(x, edge_index, W1, b1, W2, b2):
    e2d = edge_index.astype(jnp.int32).reshape(2 * DSTOFF, CH)
    degp = _deg_sc(e2d)
    s1, dinv = _prep_tc(x, W1, degp)
    p1 = _agg_sc(s1, e2d)
    s2 = _mid_tc(p1, s1, dinv, b1.reshape(1, D), W2)
    p2 = _agg_sc(s2, e2d)
    return _fin_tc(p2, s2, dinv, b2.reshape(1, D))


# speedup vs baseline: 60.0024x; 1.0832x over previous
import functools

import jax
import jax.numpy as jnp
from jax import lax
from jax.experimental import pallas as pl
from jax.experimental.pallas import tpu as pltpu
from jax.experimental.pallas import tpu_sc as plsc

N = 10000
E = 320000
D_IN = 128
D = 16

NC = 2
NS = 16
NW = NC * NS
CH = 80
EPW = E // NW
NCH = EPW // CH
DSTOFF = E // CH
ACC_ROWS = 10112
RPS = ACC_ROWS // NS

_sc_mesh = plsc.VectorSubcoreMesh(core_axis_name="c", subcore_axis_name="s")
_sc_params = pltpu.CompilerParams(use_tc_tiling_on_sc=False)


@functools.partial(
    pl.kernel,
    out_type=jax.ShapeDtypeStruct((NC, ACC_ROWS, D), jnp.float32),
    mesh=_sc_mesh,
    scratch_types=[
        pltpu.VMEM((NCH, CH), jnp.int32),
        pltpu.VMEM((CH, D), jnp.float32),
        pltpu.VMEM((RPS, D), jnp.float32),
        pltpu.VMEM_SHARED((ACC_ROWS, D), jnp.float32),
        pltpu.SemaphoreType.DMA,
    ],
    compiler_params=_sc_params,
)
def _deg_sc(e2d_hbm, out_hbm, didx, ones, stage, acc, sem):
    c = lax.axis_index("c")
    s = lax.axis_index("s")
    wid = c * NS + s
    z16 = jnp.zeros((D,), jnp.float32)
    o16 = jnp.ones((D,), jnp.float32)

    def fill(i, _):
        stage[i, :] = z16
        return 0

    lax.fori_loop(0, RPS, fill, 0)

    def fill1(i, _):
        ones[i, :] = o16
        return 0

    lax.fori_loop(0, CH, fill1, 0)
    pltpu.sync_copy(stage, acc.at[pl.ds(s * RPS, RPS)])
    pltpu.sync_copy(e2d_hbm.at[pl.ds(DSTOFF + wid * NCH, NCH)], didx)
    plsc.subcore_barrier()

    def body(i, _):
        pltpu.async_copy(ones, acc.at[didx.at[i]], sem, add=True)
        return 0

    lax.fori_loop(0, NCH, body, 0)

    def drain(i, _):
        pltpu.make_async_copy(ones, acc.at[didx.at[0]], sem).wait()
        return 0

    lax.fori_loop(0, NCH, drain, 0)
    plsc.subcore_barrier()
    r0 = s * RPS
    pltpu.sync_copy(acc.at[pl.ds(r0, RPS)], out_hbm.at[c, pl.ds(r0, RPS)])


@functools.partial(
    pl.kernel,
    out_type=jax.ShapeDtypeStruct((NC, ACC_ROWS, D), jnp.float32),
    mesh=_sc_mesh,
    scratch_types=[
        pltpu.VMEM((NCH, CH), jnp.int32),
        pltpu.VMEM((NCH, CH), jnp.int32),
        pltpu.VMEM((CH, D), jnp.float32),
        pltpu.VMEM((CH, D), jnp.float32),
        pltpu.VMEM((RPS, D), jnp.float32),
        pltpu.VMEM_SHARED((ACC_ROWS, D), jnp.float32),
        pltpu.VMEM_SHARED((ACC_ROWS, D), jnp.float32),
        pltpu.SemaphoreType.DMA,
        pltpu.SemaphoreType.DMA,
        pltpu.SemaphoreType.DMA,
        pltpu.SemaphoreType.DMA,
    ],
    compiler_params=_sc_params,
)
def _agg_sc(s_hbm, e2d_hbm, out_hbm, sidx, didx, rows0, rows1,
            stage, tbl, acc, gs0, gs1, ss0, ss1):
    c = lax.axis_index("c")
    s = lax.axis_index("s")
    wid = c * NS + s
    z16 = jnp.zeros((D,), jnp.float32)
    r0 = s * RPS

    pltpu.sync_copy(s_hbm.at[pl.ds(r0, RPS)], tbl.at[pl.ds(r0, RPS)])

    def fill(i, _):
        stage[i, :] = z16
        return 0

    lax.fori_loop(0, RPS, fill, 0)
    pltpu.sync_copy(stage, acc.at[pl.ds(r0, RPS)])

    pltpu.sync_copy(e2d_hbm.at[pl.ds(wid * NCH, NCH)], sidx)
    pltpu.sync_copy(e2d_hbm.at[pl.ds(DSTOFF + wid * NCH, NCH)], didx)
    plsc.subcore_barrier()

    def wg0():
        pltpu.make_async_copy(tbl.at[sidx.at[0]], rows0, gs0).wait()

    def wg1():
        pltpu.make_async_copy(tbl.at[sidx.at[0]], rows1, gs1).wait()

    def ws0():
        pltpu.make_async_copy(rows0, acc.at[didx.at[0]], ss0).wait()

    def ws1():
        pltpu.make_async_copy(rows1, acc.at[didx.at[0]], ss1).wait()

    pltpu.async_copy(tbl.at[sidx.at[0]], rows0, gs0)
    wg0()
    pltpu.async_copy(rows0, acc.at[didx.at[0]], ss0, add=True)
    pltpu.async_copy(tbl.at[sidx.at[1]], rows1, gs1)

    def body(j, _):
        i0 = 1 + 2 * j
        ws0()
        pltpu.async_copy(tbl.at[sidx.at[i0 + 1]], rows0, gs0)
        wg1()
        pltpu.async_copy(rows1, acc.at[didx.at[i0]], ss1, add=True)
        i1 = i0 + 1
        ws1()
        pltpu.async_copy(tbl.at[sidx.at[i1 + 1]], rows1, gs1)
        wg0()
        pltpu.async_copy(rows0, acc.at[didx.at[i1]], ss0, add=True)
        return 0

    lax.fori_loop(0, (NCH - 3) // 2, body, 0)
    ws0()
    pltpu.async_copy(tbl.at[sidx.at[NCH - 1]], rows0, gs0)
    wg1()
    pltpu.async_copy(rows1, acc.at[didx.at[NCH - 2]], ss1, add=True)
    wg0()
    pltpu.async_copy(rows0, acc.at[didx.at[NCH - 1]], ss0, add=True)
    ws1()
    ws0()

    plsc.subcore_barrier()
    pltpu.sync_copy(acc.at[pl.ds(r0, RPS)], out_hbm.at[c, pl.ds(r0, RPS)])


def _prep_body(x_ref, w1_ref, degp_ref, s1_ref, dinv_ref):
    deg = degp_ref[0, :N, 0:1] + degp_ref[1, :N, 0:1] + 1.0
    dinv = lax.rsqrt(deg)
    xw = jnp.dot(x_ref[...], w1_ref[...], preferred_element_type=jnp.float32)
    s1_ref[:N, :] = xw * dinv
    s1_ref[N:, :] = jnp.zeros((ACC_ROWS - N, D), jnp.float32)
    dinv_ref[...] = dinv


def _mid_body(p_ref, s1_ref, dinv_ref, b1_ref, w2_ref, s2_ref):
    dinv = dinv_ref[...]
    agg = p_ref[0, :N, :] + p_ref[1, :N, :] + s1_ref[:N, :]
    h = jnp.maximum(agg * dinv + b1_ref[...], 0.0)
    s2_ref[:N, :] = jnp.dot(h, w2_ref[...], preferred_element_type=jnp.float32) * dinv
    s2_ref[N:, :] = jnp.zeros((ACC_ROWS - N, D), jnp.float32)


def _fin_body(p_ref, s2_ref, dinv_ref, b2_ref, o_ref):
    agg = p_ref[0, :N, :] + p_ref[1, :N, :] + s2_ref[:N, :]
    o_ref[...] = agg * dinv_ref[...] + b2_ref[...]


_prep_tc = pl.pallas_call(
    _prep_body,
    out_shape=[
        jax.ShapeDtypeStruct((ACC_ROWS, D), jnp.float32),
        jax.ShapeDtypeStruct((N, 1), jnp.float32),
    ],
)

_mid_tc = pl.pallas_call(
    _mid_body,
    out_shape=jax.ShapeDtypeStruct((ACC_ROWS, D), jnp.float32),
)

_fin_tc = pl.pallas_call(
    _fin_body,
    out_shape=jax.ShapeDtypeStruct((N, D), jnp.float32),
)


def kernel(x, edge_index, W1, b1, W2, b2):
    e2d = edge_index.astype(jnp.int32).reshape(2 * DSTOFF, CH)
    degp = _deg_sc(e2d)
    s1, dinv = _prep_tc(x, W1, degp)
    p1 = _agg_sc(s1, e2d)
    s2 = _mid_tc(p1, s1, dinv, b1.reshape(1, D), W2)
    p2 = _agg_sc(s2, e2d)
    return _fin_tc(p2, s2, dinv, b2.reshape(1, D))


# speedup vs baseline: 86.1352x; 1.4355x over previous
import functools

import jax
import jax.numpy as jnp
from jax import lax
from jax.experimental import pallas as pl
from jax.experimental.pallas import tpu as pltpu
from jax.experimental.pallas import tpu_sc as plsc

N = 10000
E = 320000
D_IN = 128
D = 16

NC = 2
NS = 16
NW = NC * NS
CH = 80
EPW = E // NW
NCH = EPW // CH
DSTOFF = E // CH
ACC_ROWS = 10112
RPS = ACC_ROWS // NS

_sc_mesh = plsc.VectorSubcoreMesh(core_axis_name="c", subcore_axis_name="s")
_sc_params = pltpu.CompilerParams(use_tc_tiling_on_sc=False)


@functools.partial(
    pl.kernel,
    out_type=jax.ShapeDtypeStruct((NC, ACC_ROWS, D), jnp.float32),
    mesh=_sc_mesh,
    scratch_types=[
        pltpu.VMEM((NCH, CH), jnp.int32),
        pltpu.VMEM((CH, D), jnp.float32),
        pltpu.VMEM((RPS, D), jnp.float32),
        pltpu.VMEM_SHARED((ACC_ROWS, D), jnp.float32),
        pltpu.SemaphoreType.DMA,
    ],
    compiler_params=_sc_params,
)
def _deg_sc(e2d_hbm, out_hbm, didx, ones, stage, acc, sem):
    c = lax.axis_index("c")
    s = lax.axis_index("s")
    wid = c * NS + s
    z16 = jnp.zeros((D,), jnp.float32)
    o16 = jnp.ones((D,), jnp.float32)

    def fill(i, _):
        stage[i, :] = z16
        return 0

    lax.fori_loop(0, RPS, fill, 0)

    def fill1(i, _):
        ones[i, :] = o16
        return 0

    lax.fori_loop(0, CH, fill1, 0)
    pltpu.sync_copy(stage, acc.at[pl.ds(s * RPS, RPS)])
    pltpu.sync_copy(e2d_hbm.at[pl.ds(DSTOFF + wid * NCH, NCH)], didx)
    plsc.subcore_barrier()

    def body(i, _):
        pltpu.async_copy(ones, acc.at[didx.at[i]], sem, add=True)
        return 0

    lax.fori_loop(0, NCH, body, 0)

    def drain(i, _):
        pltpu.make_async_copy(ones, acc.at[didx.at[0]], sem).wait()
        return 0

    lax.fori_loop(0, NCH, drain, 0)
    plsc.subcore_barrier()
    r0 = s * RPS
    pltpu.sync_copy(acc.at[pl.ds(r0, RPS)], out_hbm.at[c, pl.ds(r0, RPS)])


@functools.partial(
    pl.kernel,
    out_type=jax.ShapeDtypeStruct((NC, ACC_ROWS, D), jnp.float32),
    mesh=_sc_mesh,
    scratch_types=[
        pltpu.VMEM((NCH, CH), jnp.int32),
        pltpu.VMEM((NCH, CH), jnp.int32),
        pltpu.VMEM((CH, D), jnp.float32),
        pltpu.VMEM((CH, D), jnp.float32),
        pltpu.VMEM((RPS, D), jnp.float32),
        pltpu.VMEM_SHARED((ACC_ROWS, D), jnp.float32),
        pltpu.VMEM_SHARED((ACC_ROWS, D), jnp.float32),
        pltpu.SemaphoreType.DMA,
        pltpu.SemaphoreType.DMA,
        pltpu.SemaphoreType.DMA,
        pltpu.SemaphoreType.DMA,
    ],
    compiler_params=_sc_params,
)
def _agg_sc(s_hbm, e2d_hbm, out_hbm, sidx, didx, rows0, rows1,
            stage, tbl, acc, gs0, gs1, ss0, ss1):
    c = lax.axis_index("c")
    s = lax.axis_index("s")
    wid = c * NS + s
    z16 = jnp.zeros((D,), jnp.float32)
    r0 = s * RPS

    pltpu.sync_copy(s_hbm.at[pl.ds(r0, RPS)], tbl.at[pl.ds(r0, RPS)])

    def fill(i, _):
        stage[i, :] = z16
        return 0

    lax.fori_loop(0, RPS, fill, 0)
    pltpu.sync_copy(stage, acc.at[pl.ds(r0, RPS)])

    pltpu.sync_copy(e2d_hbm.at[pl.ds(wid * NCH, NCH)], sidx)
    pltpu.sync_copy(e2d_hbm.at[pl.ds(DSTOFF + wid * NCH, NCH)], didx)
    plsc.subcore_barrier()

    def wg0():
        pltpu.make_async_copy(tbl.at[sidx.at[0]], rows0, gs0).wait()

    def wg1():
        pltpu.make_async_copy(tbl.at[sidx.at[0]], rows1, gs1).wait()

    def ws0():
        pltpu.make_async_copy(rows0, acc.at[didx.at[0]], ss0).wait()

    def ws1():
        pltpu.make_async_copy(rows1, acc.at[didx.at[0]], ss1).wait()

    pltpu.async_copy(tbl.at[sidx.at[0]], rows0, gs0)
    wg0()
    pltpu.async_copy(rows0, acc.at[didx.at[0]], ss0, add=True)
    pltpu.async_copy(tbl.at[sidx.at[1]], rows1, gs1)

    def body(j, _):
        i0 = 1 + 2 * j
        ws0()
        pltpu.async_copy(tbl.at[sidx.at[i0 + 1]], rows0, gs0)
        wg1()
        pltpu.async_copy(rows1, acc.at[didx.at[i0]], ss1, add=True)
        i1 = i0 + 1
        ws1()
        pltpu.async_copy(tbl.at[sidx.at[i1 + 1]], rows1, gs1)
        wg0()
        pltpu.async_copy(rows0, acc.at[didx.at[i1]], ss0, add=True)
        return 0

    lax.fori_loop(0, (NCH - 3) // 2, body, 0)
    ws0()
    pltpu.async_copy(tbl.at[sidx.at[NCH - 1]], rows0, gs0)
    wg1()
    pltpu.async_copy(rows1, acc.at[didx.at[NCH - 2]], ss1, add=True)
    wg0()
    pltpu.async_copy(rows0, acc.at[didx.at[NCH - 1]], ss0, add=True)
    ws1()
    ws0()

    plsc.subcore_barrier()
    pltpu.sync_copy(acc.at[pl.ds(r0, RPS)], out_hbm.at[c, pl.ds(r0, RPS)])


WR = ACC_ROWS * D // 128
WN = N * D // 128
XR = N * D_IN // 1024


def _prep_body(xw_ref, w1b_ref, degp_ref, s1_ref, dinv_ref):
    dinv = lax.rsqrt(degp_ref[0] + degp_ref[1] + 1.0)
    xw = jnp.dot(xw_ref[...], w1b_ref[...], preferred_element_type=jnp.float32)
    s1_ref[:WN, :] = xw * dinv[:WN, :]
    s1_ref[WN:, :] = jnp.zeros((WR - WN, 128), jnp.float32)
    dinv_ref[...] = dinv


def _mid_body(p_ref, s1_ref, dinv_ref, b1_ref, w2b_ref, s2_ref):
    dinv = dinv_ref[...]
    h = jnp.maximum((p_ref[0] + p_ref[1] + s1_ref[...]) * dinv + b1_ref[...], 0.0)
    s2_ref[...] = jnp.dot(h, w2b_ref[...], preferred_element_type=jnp.float32) * dinv


def _fin_body(p_ref, s2_ref, dinv_ref, b2_ref, o_ref):
    agg = p_ref[0, :WN, :] + p_ref[1, :WN, :] + s2_ref[:WN, :]
    o_ref[...] = agg * dinv_ref[:WN, :] + b2_ref[...]


_prep_tc = pl.pallas_call(
    _prep_body,
    out_shape=[
        jax.ShapeDtypeStruct((WR, 128), jnp.float32),
        jax.ShapeDtypeStruct((WR, 128), jnp.float32),
    ],
)

_mid_tc = pl.pallas_call(
    _mid_body,
    out_shape=jax.ShapeDtypeStruct((WR, 128), jnp.float32),
)

_fin_tc = pl.pallas_call(
    _fin_body,
    out_shape=jax.ShapeDtypeStruct((WN, 128), jnp.float32),
)


def kernel(x, edge_index, W1, b1, W2, b2):
    e2d = edge_index.astype(jnp.int32).reshape(2 * DSTOFF, CH)
    eye8 = jnp.eye(8, dtype=jnp.float32)
    w1b = jnp.kron(eye8, W1)
    w2b = jnp.kron(eye8, W2)
    b1w = jnp.tile(b1, 8).reshape(1, 128)
    b2w = jnp.tile(b2, 8).reshape(1, 128)

    degp = _deg_sc(e2d)
    s1w, dinvw = _prep_tc(x.reshape(XR, 1024), w1b,
                          degp.reshape(NC, WR, 128))
    p1 = _agg_sc(s1w.reshape(ACC_ROWS, D), e2d)
    s2w = _mid_tc(p1.reshape(NC, WR, 128), s1w, dinvw, b1w, w2b)
    p2 = _agg_sc(s2w.reshape(ACC_ROWS, D), e2d)
    ow = _fin_tc(p2.reshape(NC, WR, 128), s2w, dinvw, b2w)
    return ow.reshape(N, D)
